# baseline (speedup 1.0000x reference)
import jax
import jax.numpy as jnp
from jax.experimental import pallas as pl


def _bcast_kernel(in_ref, out_ref):
    out_ref[...] = jnp.broadcast_to(in_ref[...][None], out_ref.shape)


def kernel(x, pe_weight):
    batch = x.shape[0]
    max_len, d = pe_weight.shape
    rows = 512
    return pl.pallas_call(
        _bcast_kernel,
        grid=(max_len // rows,),
        in_specs=[pl.BlockSpec((rows, d), lambda i: (i, 0))],
        out_specs=pl.BlockSpec((batch, rows, d), lambda i: (0, i, 0)),
        out_shape=jax.ShapeDtypeStruct((batch, max_len, d), pe_weight.dtype),
    )(pe_weight)
